# Initial kernel scaffold; baseline (speedup 1.0000x reference)
import jax
import jax.numpy as jnp
from jax.experimental import pallas as pl


def kernel(x, W1, b1, W2, b2, W3, b3, Ws):
    raise NotImplementedError("write your pallas kernel here")


# speedup vs baseline: 2.6719x; 2.6719x over previous
import jax
import jax.numpy as jnp
from jax.experimental import pallas as pl

D = 2048
H = 1024
K = 64
TOPK = 8

BM = 512


def _router_body(x_ref, w1c_ref, b1c_ref, w2_ref, b2_ref, w3_ref, b3_ref, out_ref):
    x = x_ref[...]
    h = jnp.dot(x, w1c_ref[...], preferred_element_type=jnp.float32) + b1c_ref[...]
    h1 = jnp.maximum(h[:, :H], 0.0)
    xs = h[:, H:]
    h2 = jnp.maximum(
        jnp.dot(h1, w2_ref[...], preferred_element_type=jnp.float32)
        + b2_ref[...] + xs, 0.0)
    logits = (jnp.dot(h2, w3_ref[...], preferred_element_type=jnp.float32)
              + b3_ref[...])

    l = logits
    iota = jax.lax.broadcasted_iota(jnp.int32, l.shape, 1)
    neg = jnp.float32(-3.4e38)
    m0 = jnp.max(l, axis=-1, keepdims=True)
    out = jnp.zeros_like(l)
    denom = jnp.zeros((l.shape[0], 1), jnp.float32)
    for _ in range(TOPK):
        m = jnp.max(l, axis=-1, keepdims=True)
        first = jnp.min(jnp.where(l == m, iota, K), axis=-1, keepdims=True)
        onehot = iota == first
        e = jnp.exp(m - m0)
        out = out + jnp.where(onehot, e, 0.0)
        denom = denom + e
        l = jnp.where(onehot, neg, l)
    out_ref[...] = out / denom


def kernel(x, W1, b1, W2, b2, W3, b3, Ws):
    B = x.shape[0]
    w1c = jnp.concatenate([W1, Ws], axis=1)
    b1c = jnp.concatenate([b1, jnp.zeros_like(b1)])[None, :]
    return pl.pallas_call(
        _router_body,
        grid=(B // BM,),
        in_specs=[
            pl.BlockSpec((BM, D), lambda i: (i, 0)),
            pl.BlockSpec((D, 2 * H), lambda i: (0, 0)),
            pl.BlockSpec((1, 2 * H), lambda i: (0, 0)),
            pl.BlockSpec((H, H), lambda i: (0, 0)),
            pl.BlockSpec((1, H), lambda i: (0, 0)),
            pl.BlockSpec((H, K), lambda i: (0, 0)),
            pl.BlockSpec((1, K), lambda i: (0, 0)),
        ],
        out_specs=pl.BlockSpec((BM, K), lambda i: (i, 0)),
        out_shape=jax.ShapeDtypeStruct((B, K), jnp.float32),
    )(x, w1c, b1c, W2, b2[None, :], W3, b3[None, :])
